# Initial kernel scaffold; baseline (speedup 1.0000x reference)
import jax
import jax.numpy as jnp
from jax.experimental import pallas as pl


def kernel(y, adj):
    raise NotImplementedError("write your pallas kernel here")


# speedup vs baseline: 1.0471x; 1.0471x over previous
import jax
import jax.numpy as jnp
from jax.experimental import pallas as pl
from jax.experimental.pallas import tpu as pltpu

_NUM_LAYERS = 20
_ALPHA = 0.5
_N = 4096
_F = 16
_BM = 512
_M_BLOCKS = _N // _BM


def _lp_body(y_ref, adj_ref, out_ref, buf_ref):
    l = pl.program_id(0)
    m = pl.program_id(1)

    @pl.when(jnp.logical_and(l == 0, m == 0))
    def _init():
        buf_ref[0] = y_ref[...]

    prev = buf_ref[l % 2]
    a = adj_ref[...]
    res = (1.0 - _ALPHA) * y_ref[pl.ds(m * _BM, _BM), :]
    new = _ALPHA * jnp.dot(a, prev, preferred_element_type=jnp.float32) + res
    new = jnp.clip(new, 0.0, 1.0)
    buf_ref[(l + 1) % 2, pl.ds(m * _BM, _BM), :] = new

    @pl.when(l == _NUM_LAYERS - 1)
    def _emit():
        out_ref[pl.ds(m * _BM, _BM), :] = new


def kernel(y, adj):
    return pl.pallas_call(
        _lp_body,
        grid=(_NUM_LAYERS, _M_BLOCKS),
        in_specs=[
            pl.BlockSpec((_N, _F), lambda l, m: (0, 0)),
            pl.BlockSpec((_BM, _N), lambda l, m: (m, 0)),
        ],
        out_specs=pl.BlockSpec((_N, _F), lambda l, m: (0, 0)),
        out_shape=jax.ShapeDtypeStruct((_N, _F), jnp.float32),
        scratch_shapes=[pltpu.VMEM((2, _N, _F), jnp.float32)],
        compiler_params=pltpu.CompilerParams(
            dimension_semantics=("arbitrary", "arbitrary"),
            vmem_limit_bytes=128 * 1024 * 1024,
        ),
    )(y, adj)


# speedup vs baseline: 3.0328x; 2.8962x over previous
import jax
import jax.numpy as jnp
from jax.experimental import pallas as pl
from jax.experimental.pallas import tpu as pltpu

_NUM_LAYERS = 20
_ALPHA = 0.5
_N = 4096
_F = 16
_BM = 512
_M_BLOCKS = _N // _BM


def _lp_body(y_ref, adj_ref, out_ref, adjt_ref, buf_ref, rest_ref):
    l = pl.program_id(0)
    m = pl.program_id(1)

    @pl.when(l == 0)
    def _load():
        a = adj_ref[...]
        adjt_ref[m] = jnp.swapaxes(a, 0, 1).astype(jnp.bfloat16)

    @pl.when(jnp.logical_and(l == 0, m == 0))
    def _init():
        yt = jnp.swapaxes(y_ref[...], 0, 1)
        for mb in range(_M_BLOCKS):
            blk = yt[:, mb * _BM:(mb + 1) * _BM]
            buf_ref[0, mb] = blk.astype(jnp.bfloat16)
            rest_ref[mb] = (1.0 - _ALPHA) * blk

    @pl.when(l > 0)
    def _prop():
        p = (l + 1) % 2
        acc = jnp.zeros((_F, _BM), jnp.float32)
        for kb in range(_M_BLOCKS):
            acc += jnp.dot(
                buf_ref[p, kb],
                adjt_ref[m, pl.ds(kb * _BM, _BM), :],
                preferred_element_type=jnp.float32,
            )
        new = jnp.clip(_ALPHA * acc + rest_ref[m], 0.0, 1.0)
        buf_ref[l % 2, m] = new.astype(jnp.bfloat16)

        @pl.when(l == _NUM_LAYERS)
        def _emit():
            out_ref[...] = jnp.swapaxes(new, 0, 1)


def kernel(y, adj):
    return pl.pallas_call(
        _lp_body,
        grid=(_NUM_LAYERS + 1, _M_BLOCKS),
        in_specs=[
            pl.BlockSpec((_N, _F), lambda l, m: (0, 0)),
            pl.BlockSpec(
                (_BM, _N),
                lambda l, m: (jnp.where(l == 0, m, _M_BLOCKS - 1), 0),
            ),
        ],
        out_specs=pl.BlockSpec((_BM, _F), lambda l, m: (m, 0)),
        out_shape=jax.ShapeDtypeStruct((_N, _F), jnp.float32),
        scratch_shapes=[
            pltpu.VMEM((_M_BLOCKS, _N, _BM), jnp.bfloat16),
            pltpu.VMEM((2, _M_BLOCKS, _F, _BM), jnp.bfloat16),
            pltpu.VMEM((_M_BLOCKS, _F, _BM), jnp.float32),
        ],
        compiler_params=pltpu.CompilerParams(
            dimension_semantics=("arbitrary", "arbitrary"),
            vmem_limit_bytes=128 * 1024 * 1024,
        ),
    )(y, adj)
